# baseline (speedup 1.0000x reference)
import functools
import math

import jax
import jax.numpy as jnp
from jax import lax
from jax.experimental import pallas as pl
from jax.experimental.pallas import tpu as pltpu
from jax.experimental.pallas import tpu_sc as plsc

_HID = 768
_L = 512
_BS = 4
_BSF = 10
_NH = 12
_HD = 64
_FF = 3072
_ALPA = 0.6
_DC = 64

_NW = 32
_WTOT = _BS * _L + _BS * _DC * _BSF
_WPW = _WTOT // _NW
_PTOT = _BS * _L
_PPW = _PTOT // _NW


def _sc_gather(wemb, widx):
    n = widx.shape[0]
    npw = n // _NW
    mesh = plsc.VectorSubcoreMesh(core_axis_name="c", subcore_axis_name="s")

    @functools.partial(
        pl.kernel,
        out_type=jax.ShapeDtypeStruct((n, _HID), jnp.float32),
        mesh=mesh,
        scratch_types=[
            pltpu.VMEM((npw,), jnp.int32),
            pltpu.VMEM((npw, _HID), jnp.float32),
            pltpu.SemaphoreType.DMA,
        ],
    )
    def k(wemb_h, widx_h, wout_h, widx_v, rows_v, sem):
        wid = lax.axis_index("s") * 2 + lax.axis_index("c")
        wb = wid * npw
        pltpu.sync_copy(widx_h.at[pl.ds(wb, npw)], widx_v)
        pltpu.async_copy(wemb_h.at[widx_v], rows_v, sem).wait()
        pltpu.sync_copy(rows_v, wout_h.at[pl.ds(wb, npw)])

    return k(wemb, widx)


def _mmt(x, w):
    return lax.dot_general(x, w, (((1,), (1,)), ((), ())),
                           preferred_element_type=jnp.float32)


def _mmb(a, b):
    return jnp.dot(a, b, preferred_element_type=jnp.float32)


def _ln(x, g, b):
    m = jnp.mean(x, axis=-1, keepdims=True)
    v = jnp.mean((x - m) * (x - m), axis=-1, keepdims=True)
    return (x - m) / jnp.sqrt(v + 1e-5) * g + b


def _softmax(x):
    m = jnp.max(x, axis=-1, keepdims=True)
    e = jnp.exp(x - m)
    return e / jnp.sum(e, axis=-1, keepdims=True)


def _gru_body(x_ref, wih_ref, whh_ref, bih_ref, bhh_ref, out_ref):
    n = x_ref.shape[0]
    wih = wih_ref[...]
    whh = whh_ref[...]
    bih = bih_ref[...]
    bhh = bhh_ref[...]
    h = jnp.zeros((n, _HID), jnp.float32)
    for t in range(_BSF):
        x = x_ref[:, t, :]
        gi = _mmt(x, wih) + bih
        gh = _mmt(h, whh) + bhh
        r = jax.nn.sigmoid(gi[:, :_HID] + gh[:, :_HID])
        z = jax.nn.sigmoid(gi[:, _HID:2 * _HID] + gh[:, _HID:2 * _HID])
        nn = jnp.tanh(gi[:, 2 * _HID:] + r * gh[:, 2 * _HID:])
        h = (1.0 - z) * nn + z * h
    out_ref[...] = h


def _run_gru(dfg_emb, wih, whh, bih, bhh):
    n = dfg_emb.shape[0]
    return pl.pallas_call(
        _gru_body,
        out_shape=jax.ShapeDtypeStruct((n, _HID), jnp.float32),
    )(dfg_emb, wih, whh, bih, bhh)


def _fused_body(pos_all_ref, emb_ref, pe2_ref, pe01_ref, g_ref, pos64_ref,
                qw_ref, kw_ref, vw_ref, fw_ref,
                wq_ref, wk_ref, wv_ref, wo_ref,
                w1_ref, b1_ref, w2_ref, pw_ref,
                qb_ref, kb_ref, vb_ref, fb_ref,
                bq_ref, bk_ref, bv_ref, bo_ref,
                b2_ref, pb_ref,
                lneg_ref, lneb_ref, ln1g_ref, ln1b_ref, ln2g_ref, ln2b_ref,
                out_ref, ctx_ref, w1buf, w2buf, encbuf, poolbuf, wsem):
    b = pl.program_id(0)

    def _w1_copy(kk):
        return pltpu.make_async_copy(
            w1_ref.at[pl.ds(kk * _HID, _HID), :], w1buf.at[kk], wsem.at[kk])

    def _w2_copy(kk):
        return pltpu.make_async_copy(
            w2_ref.at[:, pl.ds(kk * _HID, _HID)], w2buf.at[kk],
            wsem.at[4 + kk])

    def _enc_copy(kk, ref):
        return pltpu.make_async_copy(ref, encbuf.at[kk], wsem.at[8 + kk])

    def _pool_copy():
        return pltpu.make_async_copy(pw_ref, poolbuf, wsem.at[12])

    @pl.when(b == 0)
    def _():
        for kk in range(_FF // _HID):
            _w1_copy(kk).start()
            _w2_copy(kk).start()
        for kk, ref in enumerate((wq_ref, wk_ref, wv_ref, wo_ref)):
            _enc_copy(kk, ref).start()
        _pool_copy().start()
    pos_all = pos_all_ref[...]
    dfg_len_all = jnp.sum((pos_all == 0).astype(jnp.int32), axis=1,
                          keepdims=True)
    dmax = jnp.max(dfg_len_all)
    pos_b = pos_all_ref[pl.ds(b, 1), :]
    token_f = (pos_b >= 2).astype(jnp.float32)
    didx = jnp.sum((pos_b >= 2).astype(jnp.int32))
    dlen = jnp.sum((pos_b == 0).astype(jnp.int32))

    dfg = g_ref[...] + pos64_ref[...]
    q = _mmt(dfg, qw_ref[...]) + qb_ref[...]
    k = _mmt(dfg, kw_ref[...]) + kb_ref[...]
    v = _mmt(dfg, vw_ref[...]) + vb_ref[...]
    sc = lax.dot_general(q, k, (((1,), (1,)), ((), ())),
                         preferred_element_type=jnp.float32)
    sc = sc * (1.0 / math.sqrt(_HID))
    col = lax.broadcasted_iota(jnp.int32, (_DC, _DC), 1)
    sc = jnp.where(col < dmax, sc, -jnp.inf)
    dfgo = _mmb(_softmax(sc), v)
    dfgo = _mmt(dfgo, fw_ref[...]) + fb_ref[...]

    emb = emb_ref[0]
    s = _mmb(token_f, emb)
    avg = s * (1.0 / (didx.astype(jnp.float32) + 1e-10))
    i64 = lax.broadcasted_iota(jnp.int32, (_L, _DC), 0)
    j64 = lax.broadcasted_iota(jnp.int32, (_L, _DC), 1)
    oh = (j64 == jnp.clip(i64 - didx, 0, _DC - 1)).astype(jnp.float32)
    dfg_rows = _mmb(oh, dfgo)
    irow = lax.broadcasted_iota(jnp.int32, (_L, _HID), 0)
    rel = irow - didx
    selm = ((rel >= 0) & (rel < dlen)).astype(jnp.float32)
    emb = emb * (1.0 - selm) + ((1.0 - _ALPA) * avg + _ALPA * dfg_rows) * selm

    prow = jnp.where(rel < 0, pe2_ref[...],
                     jnp.where(rel < dlen, pe01_ref[0:1], pe01_ref[1:2]))
    h = _ln(emb + prow, lneg_ref[...], lneb_ref[...])

    @pl.when(b == 0)
    def _():
        for kk, ref in enumerate((wq_ref, wk_ref, wv_ref, wo_ref)):
            _enc_copy(kk, ref).wait()
    qe = _mmt(h, encbuf[0]) + bq_ref[...]
    ke = _mmt(h, encbuf[1]) + bk_ref[...]
    ve = _mmt(h, encbuf[2]) + bv_ref[...]
    addm = jnp.where(pos_b != 1, 0.0, -1e9)
    for hh in range(_NH):
        sl = slice(hh * _HD, (hh + 1) * _HD)
        s2 = lax.dot_general(qe[:, sl], ke[:, sl], (((1,), (1,)), ((), ())),
                             preferred_element_type=jnp.float32)
        s2 = s2 * (1.0 / math.sqrt(_HD)) + addm
        ctx_ref[:, sl] = _mmb(_softmax(s2), ve[:, sl])
    attn = _mmt(ctx_ref[...], encbuf[3]) + bo_ref[...]
    h = _ln(h + attn, ln1g_ref[...], ln1b_ref[...])

    @pl.when(b == 0)
    def _():
        for kk in range(_FF // _HID):
            _w1_copy(kk).wait()
            _w2_copy(kk).wait()
    ff = b2_ref[...]
    for kk in range(_FF // _HID):
        gk = jax.nn.gelu(_mmt(h, w1buf[kk])
                         + b1_ref[:, kk * _HID:(kk + 1) * _HID])
        ff = ff + _mmt(gk, w2buf[kk])
    h = _ln(h + ff, ln2g_ref[...], ln2b_ref[...])

    @pl.when(b == 0)
    def _():
        _pool_copy().wait()
    pooled = jnp.tanh(_mmt(h[0:1, :], poolbuf[...]) + pb_ref[...])
    out_ref[0] = pooled


def _run_fused(pos_all, inputs_emb, pe2, pe01, g, pos64, p):
    full = lambda shape: pl.BlockSpec(shape, lambda b: (0,) * len(shape))
    r2 = lambda x: x.reshape(1, -1)
    call = pl.pallas_call(
        _fused_body,
        grid=(_BS,),
        in_specs=[
            full((_BS, _L)),
            pl.BlockSpec((1, _L, _HID), lambda b: (b, 0, 0)),
            full((_L, _HID)),
            full((2, _HID)),
            pl.BlockSpec((_DC, _HID), lambda b: (b, 0)),
            full((_DC, _HID)),
            full((_HID, _HID)), full((_HID, _HID)),
            full((_HID, _HID)), full((_HID, _HID)),
            pl.BlockSpec(memory_space=pltpu.MemorySpace.HBM),
            pl.BlockSpec(memory_space=pltpu.MemorySpace.HBM),
            pl.BlockSpec(memory_space=pltpu.MemorySpace.HBM),
            pl.BlockSpec(memory_space=pltpu.MemorySpace.HBM),
            pl.BlockSpec(memory_space=pltpu.MemorySpace.HBM),
            full((1, _FF)),
            pl.BlockSpec(memory_space=pltpu.MemorySpace.HBM),
            pl.BlockSpec(memory_space=pltpu.MemorySpace.HBM),
        ] + [full((1, _HID))] * 16,
        out_specs=pl.BlockSpec((1, 1, _HID), lambda b: (b, 0, 0)),
        out_shape=jax.ShapeDtypeStruct((_BS, 1, _HID), jnp.float32),
        scratch_shapes=[pltpu.VMEM((_L, _HID), jnp.float32),
                        pltpu.VMEM((_FF // _HID, _HID, _HID), jnp.float32),
                        pltpu.VMEM((_FF // _HID, _HID, _HID), jnp.float32),
                        pltpu.VMEM((4, _HID, _HID), jnp.float32),
                        pltpu.VMEM((_HID, _HID), jnp.float32),
                        pltpu.SemaphoreType.DMA((13,))],
    )
    return call(pos_all, inputs_emb, pe2, pe01, g, pos64,
                p['Qw'], p['Kw'], p['Vw'], p['ffw'],
                p['enc_Wq'], p['enc_Wk'], p['enc_Wv'], p['enc_Wo'],
                p['enc_W1'], r2(p['enc_b1']), p['enc_W2'], p['pool_W'],
                r2(p['Qb']), r2(p['Kb']), r2(p['Vb']), r2(p['ffb']),
                r2(p['enc_bq']), r2(p['enc_bk']), r2(p['enc_bv']),
                r2(p['enc_bo']), r2(p['enc_b2']), r2(p['pool_b']),
                r2(p['ln_emb_g']), r2(p['ln_emb_b']),
                r2(p['ln1_g']), r2(p['ln1_b']),
                r2(p['ln2_g']), r2(p['ln2_b']))


def kernel(code_inputs, attn_mask, position_idx, New_DFG_ids, params):
    p = params
    pos = position_idx.astype(jnp.int32)
    code = code_inputs.astype(jnp.int32)
    nd = New_DFG_ids.astype(jnp.int32)

    dfg_index = jnp.sum((pos >= 2).astype(jnp.int32), axis=1)
    idx = jnp.clip(dfg_index[:, None]
                   + jnp.arange(_DC, dtype=jnp.int32)[None, :], 0, _L - 1)
    fin_ids = jnp.take_along_axis(nd, idx[:, :, None], axis=1)

    code_rows = _sc_gather(p['word_emb'], code.reshape(-1))
    dfg_rows = _sc_gather(p['word_emb'], fin_ids.reshape(-1))

    inputs_emb = code_rows.reshape(_BS, _L, _HID)
    dfg_emb = dfg_rows.reshape(_BS * _DC, _BSF, _HID)

    g = _run_gru(dfg_emb, p['gru_Wih'], p['gru_Whh'],
                 p['gru_bih'][None, :], p['gru_bhh'][None, :])

    pooled = _run_fused(pos, inputs_emb,
                        p['pos_emb'][2:2 + _L], p['pos_emb'][:2], g,
                        p['pos_emb'][:_DC], p)
    return pooled.reshape(_BS, _HID)


# speedup vs baseline: 1.0780x; 1.0780x over previous
import functools
import math

import jax
import jax.numpy as jnp
from jax import lax
from jax.experimental import pallas as pl
from jax.experimental.pallas import tpu as pltpu
from jax.experimental.pallas import tpu_sc as plsc

_HID = 768
_L = 512
_BS = 4
_BSF = 10
_NH = 12
_HD = 64
_FF = 3072
_ALPA = 0.6
_DC = 64

_NW = 32
_WTOT = _BS * _L + _BS * _DC * _BSF
_WPW = _WTOT // _NW
_PTOT = _BS * _L
_PPW = _PTOT // _NW


def _sc_gather(wemb, widx):
    n = widx.shape[0]
    npw = n // _NW
    mesh = plsc.VectorSubcoreMesh(core_axis_name="c", subcore_axis_name="s")

    @functools.partial(
        pl.kernel,
        out_type=jax.ShapeDtypeStruct((n, _HID), jnp.float32),
        mesh=mesh,
        scratch_types=[
            pltpu.VMEM((npw,), jnp.int32),
            pltpu.VMEM((npw, _HID), jnp.float32),
            pltpu.SemaphoreType.DMA,
        ],
    )
    def k(wemb_h, widx_h, wout_h, widx_v, rows_v, sem):
        wid = lax.axis_index("s") * 2 + lax.axis_index("c")
        wb = wid * npw
        pltpu.sync_copy(widx_h.at[pl.ds(wb, npw)], widx_v)
        pltpu.async_copy(wemb_h.at[widx_v], rows_v, sem).wait()
        pltpu.sync_copy(rows_v, wout_h.at[pl.ds(wb, npw)])

    return k(wemb, widx)


def _mmt(x, w):
    return lax.dot_general(x, w, (((1,), (1,)), ((), ())),
                           preferred_element_type=jnp.float32)


def _mmb(a, b):
    return jnp.dot(a, b, preferred_element_type=jnp.float32)


def _ln(x, g, b):
    m = jnp.mean(x, axis=-1, keepdims=True)
    v = jnp.mean((x - m) * (x - m), axis=-1, keepdims=True)
    return (x - m) / jnp.sqrt(v + 1e-5) * g + b


def _softmax(x):
    m = jnp.max(x, axis=-1, keepdims=True)
    e = jnp.exp(x - m)
    return e / jnp.sum(e, axis=-1, keepdims=True)


def _gru_body(x_ref, wih_ref, whh_ref, bih_ref, bhh_ref, out_ref):
    n = x_ref.shape[0]
    wih = wih_ref[...]
    whh = whh_ref[...]
    bih = bih_ref[...]
    bhh = bhh_ref[...]
    h = jnp.zeros((n, _HID), jnp.float32)
    for t in range(_BSF):
        x = x_ref[:, t, :]
        gi = _mmt(x, wih) + bih
        gh = _mmt(h, whh) + bhh
        r = jax.nn.sigmoid(gi[:, :_HID] + gh[:, :_HID])
        z = jax.nn.sigmoid(gi[:, _HID:2 * _HID] + gh[:, _HID:2 * _HID])
        nn = jnp.tanh(gi[:, 2 * _HID:] + r * gh[:, 2 * _HID:])
        h = (1.0 - z) * nn + z * h
    out_ref[...] = h


def _run_gru(dfg_emb, wih, whh, bih, bhh):
    n = dfg_emb.shape[0]
    return pl.pallas_call(
        _gru_body,
        out_shape=jax.ShapeDtypeStruct((n, _HID), jnp.float32),
    )(dfg_emb, wih, whh, bih, bhh)


def _fused_body(pos_all_ref, emb_ref, pe2_ref, pe01_ref, g_ref, pos64_ref,
                qw_ref, kw_ref, vw_ref, fw_ref,
                wq_ref, wk_ref, wv_ref, wo_ref,
                w1_ref, b1_ref, w2_ref, pw_ref,
                qb_ref, kb_ref, vb_ref, fb_ref,
                bq_ref, bk_ref, bv_ref, bo_ref,
                b2_ref, pb_ref,
                lneg_ref, lneb_ref, ln1g_ref, ln1b_ref, ln2g_ref, ln2b_ref,
                out_ref, ctx_ref, w1buf, w2buf, wsem):
    b = pl.program_id(0)

    def _w1_copy(kk):
        return pltpu.make_async_copy(
            w1_ref.at[pl.ds(kk * _HID, _HID), :], w1buf.at[kk], wsem.at[kk])

    def _w2_copy(kk):
        return pltpu.make_async_copy(
            w2_ref.at[:, pl.ds(kk * _HID, _HID)], w2buf.at[kk],
            wsem.at[4 + kk])

    @pl.when(b == 0)
    def _():
        for kk in range(_FF // _HID):
            _w1_copy(kk).start()
            _w2_copy(kk).start()
    pos_all = pos_all_ref[...]
    dfg_len_all = jnp.sum((pos_all == 0).astype(jnp.int32), axis=1,
                          keepdims=True)
    dmax = jnp.max(dfg_len_all)
    pos_b = pos_all_ref[pl.ds(b, 1), :]
    token_f = (pos_b >= 2).astype(jnp.float32)
    didx = jnp.sum((pos_b >= 2).astype(jnp.int32))
    dlen = jnp.sum((pos_b == 0).astype(jnp.int32))

    dfg = g_ref[...] + pos64_ref[...]
    q = _mmt(dfg, qw_ref[...]) + qb_ref[...]
    k = _mmt(dfg, kw_ref[...]) + kb_ref[...]
    v = _mmt(dfg, vw_ref[...]) + vb_ref[...]
    sc = lax.dot_general(q, k, (((1,), (1,)), ((), ())),
                         preferred_element_type=jnp.float32)
    sc = sc * (1.0 / math.sqrt(_HID))
    col = lax.broadcasted_iota(jnp.int32, (_DC, _DC), 1)
    sc = jnp.where(col < dmax, sc, -jnp.inf)
    dfgo = _mmb(_softmax(sc), v)
    dfgo = _mmt(dfgo, fw_ref[...]) + fb_ref[...]

    emb = emb_ref[0]
    s = _mmb(token_f, emb)
    avg = s * (1.0 / (didx.astype(jnp.float32) + 1e-10))
    i64 = lax.broadcasted_iota(jnp.int32, (_L, _DC), 0)
    j64 = lax.broadcasted_iota(jnp.int32, (_L, _DC), 1)
    oh = (j64 == jnp.clip(i64 - didx, 0, _DC - 1)).astype(jnp.float32)
    dfg_rows = _mmb(oh, dfgo)
    irow = lax.broadcasted_iota(jnp.int32, (_L, _HID), 0)
    rel = irow - didx
    selm = ((rel >= 0) & (rel < dlen)).astype(jnp.float32)
    emb = emb * (1.0 - selm) + ((1.0 - _ALPA) * avg + _ALPA * dfg_rows) * selm

    prow = jnp.where(rel < 0, pe2_ref[...],
                     jnp.where(rel < dlen, pe01_ref[0:1], pe01_ref[1:2]))
    h = _ln(emb + prow, lneg_ref[...], lneb_ref[...])
    qe = _mmt(h, wq_ref[...]) + bq_ref[...]
    ke = _mmt(h, wk_ref[...]) + bk_ref[...]
    ve = _mmt(h, wv_ref[...]) + bv_ref[...]
    addm = jnp.where(pos_b != 1, 0.0, -1e9)
    for hh in range(_NH):
        sl = slice(hh * _HD, (hh + 1) * _HD)
        s2 = lax.dot_general(qe[:, sl], ke[:, sl], (((1,), (1,)), ((), ())),
                             preferred_element_type=jnp.float32)
        s2 = s2 * (1.0 / math.sqrt(_HD)) + addm
        ctx_ref[:, sl] = _mmb(_softmax(s2), ve[:, sl])
    attn = _mmt(ctx_ref[...], wo_ref[...]) + bo_ref[...]
    h = _ln(h + attn, ln1g_ref[...], ln1b_ref[...])

    @pl.when(b == 0)
    def _():
        for kk in range(_FF // _HID):
            _w1_copy(kk).wait()
            _w2_copy(kk).wait()
    ff = b2_ref[...]
    for kk in range(_FF // _HID):
        gk = jax.nn.gelu(_mmt(h, w1buf[kk])
                         + b1_ref[:, kk * _HID:(kk + 1) * _HID])
        ff = ff + _mmt(gk, w2buf[kk])
    h = _ln(h + ff, ln2g_ref[...], ln2b_ref[...])
    pooled = jnp.tanh(_mmt(h[0:1, :], pw_ref[...]) + pb_ref[...])
    out_ref[0] = pooled


def _run_fused(pos_all, inputs_emb, pe2, pe01, g, pos64, p):
    full = lambda shape: pl.BlockSpec(shape, lambda b: (0,) * len(shape))
    r2 = lambda x: x.reshape(1, -1)
    call = pl.pallas_call(
        _fused_body,
        grid=(_BS,),
        in_specs=[
            full((_BS, _L)),
            pl.BlockSpec((1, _L, _HID), lambda b: (b, 0, 0)),
            full((_L, _HID)),
            full((2, _HID)),
            pl.BlockSpec((_DC, _HID), lambda b: (b, 0)),
            full((_DC, _HID)),
            full((_HID, _HID)), full((_HID, _HID)),
            full((_HID, _HID)), full((_HID, _HID)),
            full((_HID, _HID)), full((_HID, _HID)),
            full((_HID, _HID)), full((_HID, _HID)),
            pl.BlockSpec(memory_space=pltpu.MemorySpace.HBM),
            full((1, _FF)),
            pl.BlockSpec(memory_space=pltpu.MemorySpace.HBM),
            full((_HID, _HID)),
        ] + [full((1, _HID))] * 16,
        out_specs=pl.BlockSpec((1, 1, _HID), lambda b: (b, 0, 0)),
        out_shape=jax.ShapeDtypeStruct((_BS, 1, _HID), jnp.float32),
        scratch_shapes=[pltpu.VMEM((_L, _HID), jnp.float32),
                        pltpu.VMEM((_FF // _HID, _HID, _HID), jnp.float32),
                        pltpu.VMEM((_FF // _HID, _HID, _HID), jnp.float32),
                        pltpu.SemaphoreType.DMA((2 * _FF // _HID,))],
    )
    return call(pos_all, inputs_emb, pe2, pe01, g, pos64,
                p['Qw'], p['Kw'], p['Vw'], p['ffw'],
                p['enc_Wq'], p['enc_Wk'], p['enc_Wv'], p['enc_Wo'],
                p['enc_W1'], r2(p['enc_b1']), p['enc_W2'], p['pool_W'],
                r2(p['Qb']), r2(p['Kb']), r2(p['Vb']), r2(p['ffb']),
                r2(p['enc_bq']), r2(p['enc_bk']), r2(p['enc_bv']),
                r2(p['enc_bo']), r2(p['enc_b2']), r2(p['pool_b']),
                r2(p['ln_emb_g']), r2(p['ln_emb_b']),
                r2(p['ln1_g']), r2(p['ln1_b']),
                r2(p['ln2_g']), r2(p['ln2_b']))


def kernel(code_inputs, attn_mask, position_idx, New_DFG_ids, params):
    p = params
    pos = position_idx.astype(jnp.int32)
    code = code_inputs.astype(jnp.int32)
    nd = New_DFG_ids.astype(jnp.int32)

    dfg_index = jnp.sum((pos >= 2).astype(jnp.int32), axis=1)
    idx = jnp.clip(dfg_index[:, None]
                   + jnp.arange(_DC, dtype=jnp.int32)[None, :], 0, _L - 1)
    fin_ids = jnp.take_along_axis(nd, idx[:, :, None], axis=1)

    code_rows = _sc_gather(p['word_emb'], code.reshape(-1))
    dfg_rows = _sc_gather(p['word_emb'], fin_ids.reshape(-1))

    inputs_emb = code_rows.reshape(_BS, _L, _HID)
    dfg_emb = dfg_rows.reshape(_BS * _DC, _BSF, _HID)

    g = _run_gru(dfg_emb, p['gru_Wih'], p['gru_Whh'],
                 p['gru_bih'][None, :], p['gru_bhh'][None, :])

    pooled = _run_fused(pos, inputs_emb,
                        p['pos_emb'][2:2 + _L], p['pos_emb'][:2], g,
                        p['pos_emb'][:_DC], p)
    return pooled.reshape(_BS, _HID)
